# baseline (speedup 1.0000x reference)
import functools

import jax
import jax.numpy as jnp
from jax import lax
from jax.experimental import pallas as pl
from jax.experimental.pallas import tpu as pltpu
from jax.experimental.pallas import tpu_sc as plsc


def _make_gather(V, D, B, NC, NS):
    NW = NC * NS
    C = 128
    G = 4
    b_per_w = B // NW
    K = b_per_w // C
    S = K // G
    assert b_per_w * NW == B and K * C == b_per_w and S * G == K

    mesh = plsc.VectorSubcoreMesh(core_axis_name="c", subcore_axis_name="s")

    @functools.partial(
        pl.kernel,
        mesh=mesh,
        compiler_params=pltpu.CompilerParams(use_tc_tiling_on_sc=False),
        out_type=jax.ShapeDtypeStruct((B, D), jnp.float32),
        scratch_types=[
            pltpu.VMEM((K, C), jnp.int32),
            pltpu.VMEM((G * C, D), jnp.float32),
            pltpu.SemaphoreType.DMA,
        ],
    )
    def gather_kernel(table_hbm, idx_hbm, out_hbm, idx_v, rows_v, gsem):
        wid = lax.axis_index("s") * NC + lax.axis_index("c")
        base = wid * b_per_w
        pltpu.sync_copy(idx_hbm.at[wid], idx_v)

        @pl.loop(0, S)
        def _chunk(s):
            descs = []
            for g in range(G):
                d = pltpu.async_copy(
                    table_hbm.at[idx_v.at[s * G + g]],
                    rows_v.at[pl.ds(g * C, C)],
                    gsem,
                )
                descs.append(d)
            for d in descs:
                d.wait()
            pltpu.sync_copy(rows_v, out_hbm.at[pl.ds(base + s * (G * C), G * C)])

    return gather_kernel


def kernel(input, dim, index):
    table = input
    V, D = table.shape
    (B,) = index.shape
    info = plsc.get_sparse_core_info()
    NC, NS = info.num_cores, info.num_subcores
    NW = NC * NS
    C = 128
    idx = index.astype(jnp.int32).reshape(NW, (B // NW) // C, C)
    return _make_gather(V, D, B, NC, NS)(table, idx)


# speedup vs baseline: 1.0220x; 1.0220x over previous
import functools

import jax
import jax.numpy as jnp
from jax import lax
from jax.experimental import pallas as pl
from jax.experimental.pallas import tpu as pltpu
from jax.experimental.pallas import tpu_sc as plsc

_NBUF = 8


def _make_gather(V, D, B, NC, NS):
    NW = NC * NS
    C = 128
    b_per_w = B // NW
    K = b_per_w // C
    assert b_per_w * NW == B and K * C == b_per_w and K % _NBUF == 0

    mesh = plsc.VectorSubcoreMesh(core_axis_name="c", subcore_axis_name="s")

    @functools.partial(
        pl.kernel,
        mesh=mesh,
        compiler_params=pltpu.CompilerParams(use_tc_tiling_on_sc=False),
        out_type=jax.ShapeDtypeStruct((B, D), jnp.float32),
        scratch_types=[
            pltpu.VMEM((K, C), jnp.int32),
            [pltpu.VMEM((C, D), jnp.float32) for _ in range(_NBUF)],
            [pltpu.SemaphoreType.DMA for _ in range(_NBUF)],
            [pltpu.SemaphoreType.DMA for _ in range(_NBUF)],
        ],
    )
    def gather_kernel(table_hbm, idx_hbm, out_hbm, idx_v, bufs, gsems, wsems):
        wid = lax.axis_index("s") * NC + lax.axis_index("c")
        base = wid * b_per_w
        pltpu.sync_copy(idx_hbm.at[wid], idx_v)

        def gather(g, b):
            return pltpu.make_async_copy(
                table_hbm.at[idx_v.at[g]], bufs[b], gsems[b])

        def writeback(g, b):
            return pltpu.make_async_copy(
                bufs[b], out_hbm.at[pl.ds(base + g * C, C)], wsems[b])

        for b in range(_NBUF):
            gather(b, b).start()

        @pl.loop(0, K, step=_NBUF)
        def _lap(j):
            for b in range(_NBUF):
                g = j + b
                gather(g, b).wait()
                writeback(g, b).start()
                writeback(g, b).wait()
                @pl.when(g + _NBUF < K)
                def _refill():
                    gather(g + _NBUF, b).start()

    return gather_kernel


def kernel(input, dim, index):
    table = input
    V, D = table.shape
    (B,) = index.shape
    info = plsc.get_sparse_core_info()
    NC, NS = info.num_cores, info.num_subcores
    NW = NC * NS
    C = 128
    idx = index.astype(jnp.int32).reshape(NW, (B // NW) // C, C)
    return _make_gather(V, D, B, NC, NS)(table, idx)
